# Initial kernel scaffold; baseline (speedup 1.0000x reference)
import jax
import jax.numpy as jnp
from jax.experimental import pallas as pl


def kernel(params, sample_reps, data, labels, perm):
    raise NotImplementedError("write your pallas kernel here")


# speedup vs baseline: 4.5081x; 4.5081x over previous
import functools

import jax
import jax.numpy as jnp
from jax.experimental import pallas as pl
from jax.experimental.pallas import tpu as pltpu


def _vmem_limit(block_bytes):
    cap = (64 << 20) * 3 // 4
    return int(min(max(3 * block_bytes + (4 << 20), 16 << 20), cap))


def _center_kernel(params_ref, reps_ref, centers_ref, c2_ref):
    p = params_ref[...].astype(jnp.float32)
    p = p - jnp.max(p, axis=-1, keepdims=True)
    e = jnp.exp(p)
    w = e / jnp.sum(e, axis=-1, keepdims=True)
    reps = reps_ref[...].astype(jnp.float32)
    centers = jax.lax.dot_general(
        w[:, None, :], reps, (((2,), (1,)), ((0,), (0,))),
        preferred_element_type=jnp.float32)[:, 0, :]
    c2_ref[...] = jnp.sum(centers * centers, axis=-1, keepdims=True)
    centers_ref[...] = (2.0 * centers).astype(jnp.bfloat16)


def _loss_kernel(x_ref, labels_ref, bid_ref, centers_ref, c2_ref, out_ref):
    C = centers_ref.shape[0]
    bk = x_ref.shape[0]

    x = x_ref[...].astype(jnp.bfloat16)
    dis = jax.lax.dot_general(
        centers_ref[...], x, (((1,), (1,)), ((), ())),
        preferred_element_type=jnp.float32)
    dis = dis - c2_ref[...]

    m = jnp.max(dis, axis=0, keepdims=True)
    lse = jnp.log(jnp.sum(jnp.exp(dis - m), axis=0, keepdims=True)) + m

    row = jax.lax.broadcasted_iota(jnp.int32, (C, bk), 0)
    tgt = jnp.sum(jnp.where(row == labels_ref[0], dis, 0.0),
                  axis=0, keepdims=True)
    per_sample = lse - tgt

    part = jnp.sum(jnp.where(row == bid_ref[0], per_sample, 0.0),
                   axis=1, keepdims=True)
    out_ref[0] = part


def _finalize_kernel(part_ref, out_ref, *, batch_size):
    nb = out_ref.shape[0]
    sums = jnp.sum(part_ref[...], axis=0)
    ce = sums[:nb] * (1.0 / batch_size)
    out_ref[...] = ce * ce / (ce + 1e-7)


def _forward(params, sample_reps, data, labels, perm, batch_size):
    C, S = params.shape
    N, D = data.shape
    num_batches = N // batch_size
    rows = num_batches * batch_size

    centers2, c2 = pl.pallas_call(
        _center_kernel,
        out_shape=(jax.ShapeDtypeStruct((C, D), jnp.bfloat16),
                   jax.ShapeDtypeStruct((C, 1), jnp.float32)),
        grid=(1,),
        in_specs=[
            pl.BlockSpec((C, S), lambda i: (0, 0)),
            pl.BlockSpec((C, S, D), lambda i: (0, 0, 0)),
        ],
        out_specs=(
            pl.BlockSpec((C, D), lambda i: (0, 0)),
            pl.BlockSpec((C, 1), lambda i: (0, 0)),
        ),
        compiler_params=pltpu.CompilerParams(
            vmem_limit_bytes=_vmem_limit(C * S * (D + 1) * 4)),
    )(params, sample_reps)

    filler = jnp.int32(min(num_batches, C - 1))
    bid = jnp.full((N,), filler, jnp.int32).at[perm[:rows]].set(
        jnp.arange(rows, dtype=jnp.int32) // batch_size)

    bk = 4096
    while N % bk:
        bk //= 2
    steps = N // bk
    labels3 = labels.astype(jnp.int32).reshape(steps, 1, bk)
    bid3 = bid.reshape(steps, 1, bk)

    block_bytes = bk * D * 4 + 2 * bk * 4 + C * D * 2 + C * 4
    part = pl.pallas_call(
        _loss_kernel,
        out_shape=jax.ShapeDtypeStruct((steps, C, 1), jnp.float32),
        grid=(steps,),
        in_specs=[
            pl.BlockSpec((bk, D), lambda i: (i, 0)),
            pl.BlockSpec((1, 1, bk), lambda i: (i, 0, 0)),
            pl.BlockSpec((1, 1, bk), lambda i: (i, 0, 0)),
            pl.BlockSpec((C, D), lambda i: (0, 0)),
            pl.BlockSpec((C, 1), lambda i: (0, 0)),
        ],
        out_specs=pl.BlockSpec((1, C, 1), lambda i: (i, 0, 0)),
        compiler_params=pltpu.CompilerParams(
            dimension_semantics=("parallel",),
            vmem_limit_bytes=_vmem_limit(block_bytes)),
    )(data, labels3, bid3, centers2, c2)

    return pl.pallas_call(
        functools.partial(_finalize_kernel, batch_size=batch_size),
        out_shape=jax.ShapeDtypeStruct((num_batches, 1), jnp.float32),
        grid=(1,),
        in_specs=[pl.BlockSpec((steps, C, 1), lambda i: (0, 0, 0))],
        out_specs=pl.BlockSpec((num_batches, 1), lambda i: (0, 0)),
    )(part)


def kernel(params, sample_reps, data, labels, perm):
    return _forward(params, sample_reps, data, labels, perm, 2048)


# speedup vs baseline: 19.5840x; 4.3442x over previous
import functools

import jax
import jax.numpy as jnp
from jax.experimental import pallas as pl
from jax.experimental.pallas import tpu as pltpu


def _vmem_limit(block_bytes):
    cap = (64 << 20) * 3 // 4
    return int(min(max(3 * block_bytes + (4 << 20), 16 << 20), cap))


def _center_kernel(params_ref, reps_ref, centers_ref, c2_ref):
    p = params_ref[...].astype(jnp.float32)
    p = p - jnp.max(p, axis=-1, keepdims=True)
    e = jnp.exp(p)
    w = e / jnp.sum(e, axis=-1, keepdims=True)
    reps = reps_ref[...].astype(jnp.float32)
    centers = jax.lax.dot_general(
        w[:, None, :], reps, (((2,), (1,)), ((0,), (0,))),
        preferred_element_type=jnp.float32)[:, 0, :]
    c2_ref[...] = jnp.sum(centers * centers, axis=-1, keepdims=True)
    centers_ref[...] = (2.0 * centers).astype(jnp.bfloat16)


def _invperm_kernel(perm_ref, bidp_ref, *, pk, batch_size):
    c = pl.program_id(0)
    j = pl.program_id(1)
    nj = pl.num_programs(1)
    R = bidp_ref.shape[1]

    pv = perm_ref[0]
    hi = pv // 128
    lo = pv - hi * 128
    A = (jax.lax.broadcasted_iota(jnp.int32, (R, pk), 0)
         == hi).astype(jnp.bfloat16)
    p_glob = ((c * nj + j) * pk
              + jax.lax.broadcasted_iota(jnp.int32, (1, pk), 1))
    w = (p_glob // batch_size).astype(jnp.bfloat16)
    B = (jax.lax.broadcasted_iota(jnp.int32, (128, pk), 0)
         == lo).astype(jnp.bfloat16) * w
    C = jax.lax.dot_general(A, B, (((1,), (1,)), ((), ())),
                            preferred_element_type=jnp.float32)

    @pl.when(j == 0)
    def _init():
        bidp_ref[0] = C

    @pl.when(j > 0)
    def _acc():
        bidp_ref[0] += C


def _loss_kernel(x_ref, labels_ref, bid_ref, centers_ref, c2_ref, out_ref):
    C = centers_ref.shape[0]
    bk = x_ref.shape[0]

    x = x_ref[...].astype(jnp.bfloat16)
    dis = jax.lax.dot_general(
        centers_ref[...], x, (((1,), (1,)), ((), ())),
        preferred_element_type=jnp.float32)
    dis = dis - c2_ref[...]

    m = jnp.max(dis, axis=0, keepdims=True)
    lse = jnp.log(jnp.sum(jnp.exp(dis - m), axis=0, keepdims=True)) + m

    row = jax.lax.broadcasted_iota(jnp.int32, (C, bk), 0)
    tgt = jnp.sum(jnp.where(row == labels_ref[0], dis, 0.0),
                  axis=0, keepdims=True)
    per_sample = lse - tgt

    bid_i = bid_ref[0].astype(jnp.int32)
    part = jnp.sum(jnp.where(row == bid_i, per_sample, 0.0),
                   axis=1, keepdims=True)
    out_ref[0] = part


def _finalize_kernel(part_ref, out_ref, *, batch_size):
    nb = out_ref.shape[0]
    sums = jnp.sum(part_ref[...], axis=0)
    ce = sums[:nb] * (1.0 / batch_size)
    out_ref[...] = ce * ce / (ce + 1e-7)


def _forward(params, sample_reps, data, labels, perm, batch_size):
    C, S = params.shape
    N, D = data.shape
    num_batches = N // batch_size
    rows = num_batches * batch_size

    centers2, c2 = pl.pallas_call(
        _center_kernel,
        out_shape=(jax.ShapeDtypeStruct((C, D), jnp.bfloat16),
                   jax.ShapeDtypeStruct((C, 1), jnp.float32)),
        grid=(1,),
        in_specs=[
            pl.BlockSpec((C, S), lambda i: (0, 0)),
            pl.BlockSpec((C, S, D), lambda i: (0, 0, 0)),
        ],
        out_specs=(
            pl.BlockSpec((C, D), lambda i: (0, 0)),
            pl.BlockSpec((C, 1), lambda i: (0, 0)),
        ),
        compiler_params=pltpu.CompilerParams(
            vmem_limit_bytes=_vmem_limit(C * S * (D + 1) * 4)),
    )(params, sample_reps)

    R = N // 128
    pk = 4096
    while N % pk:
        pk //= 2
    pblocks = N // pk
    cores = 2 if pblocks % 2 == 0 else 1
    inner = pblocks // cores
    perm3 = perm.astype(jnp.int32).reshape(pblocks, 1, pk)
    bidp = pl.pallas_call(
        functools.partial(_invperm_kernel, pk=pk, batch_size=batch_size),
        out_shape=jax.ShapeDtypeStruct((cores, R, 128), jnp.float32),
        grid=(cores, inner),
        in_specs=[pl.BlockSpec((1, 1, pk),
                               lambda c, j, inner=inner: (c * inner + j, 0, 0))],
        out_specs=pl.BlockSpec((1, R, 128), lambda c, j: (c, 0, 0)),
        compiler_params=pltpu.CompilerParams(
            dimension_semantics=("parallel", "arbitrary"),
            vmem_limit_bytes=_vmem_limit(R * pk * 2 + 129 * pk * 2 + R * 512)),
    )(perm3)
    bid = jnp.sum(bidp, axis=0)

    bk = 4096
    while N % bk:
        bk //= 2
    steps = N // bk
    labels3 = labels.astype(jnp.int32).reshape(steps, 1, bk)
    bid3 = bid.reshape(steps, 1, bk)

    block_bytes = bk * D * 4 + 2 * bk * 4 + C * D * 2 + C * 4
    part = pl.pallas_call(
        _loss_kernel,
        out_shape=jax.ShapeDtypeStruct((steps, C, 1), jnp.float32),
        grid=(steps,),
        in_specs=[
            pl.BlockSpec((bk, D), lambda i: (i, 0)),
            pl.BlockSpec((1, 1, bk), lambda i: (i, 0, 0)),
            pl.BlockSpec((1, 1, bk), lambda i: (i, 0, 0)),
            pl.BlockSpec((C, D), lambda i: (0, 0)),
            pl.BlockSpec((C, 1), lambda i: (0, 0)),
        ],
        out_specs=pl.BlockSpec((1, C, 1), lambda i: (i, 0, 0)),
        compiler_params=pltpu.CompilerParams(
            dimension_semantics=("parallel",),
            vmem_limit_bytes=_vmem_limit(block_bytes)),
    )(data, labels3, bid3, centers2, c2)

    return pl.pallas_call(
        functools.partial(_finalize_kernel, batch_size=batch_size),
        out_shape=jax.ShapeDtypeStruct((num_batches, 1), jnp.float32),
        grid=(1,),
        in_specs=[pl.BlockSpec((steps, C, 1), lambda i: (0, 0, 0))],
        out_specs=pl.BlockSpec((num_batches, 1), lambda i: (0, 0)),
    )(part)


def kernel(params, sample_reps, data, labels, perm):
    return _forward(params, sample_reps, data, labels, perm, 2048)
